# baseline (speedup 1.0000x reference)
import functools

import jax
import jax.numpy as jnp
from jax import lax
from jax.experimental import pallas as pl
from jax.experimental.pallas import tpu as pltpu
from jax.experimental.pallas import tpu_sc as plsc

D = 64
B = 16384
NC, NS = 2, 16
NW = NC * NS
B_PER_W = B // NW
CHUNK = 128
NCHUNK = B_PER_W // CHUNK


@functools.partial(
    pl.kernel,
    mesh=plsc.VectorSubcoreMesh(core_axis_name="c", subcore_axis_name="s"),
    out_type=jax.ShapeDtypeStruct((B, D), jnp.float32),
    scratch_types=[
        pltpu.VMEM((NCHUNK, CHUNK), jnp.int32),
        pltpu.VMEM((B_PER_W, D), jnp.float32),
        pltpu.SemaphoreType.DMA,
    ],
    compiler_params=pltpu.CompilerParams(use_tc_tiling_on_sc=False),
)
def _sc_gather(table_hbm, idx_hbm, out_hbm, idx_v, rows_v, sem):
    wid = lax.axis_index("s") * NC + lax.axis_index("c")
    base = wid * B_PER_W
    pltpu.sync_copy(idx_hbm.at[wid], idx_v)
    copies = [
        pltpu.async_copy(
            table_hbm.at[idx_v.at[j]],
            rows_v.at[pl.ds(j * CHUNK, CHUNK)],
            sem,
        )
        for j in range(NCHUNK)
    ]
    for c in copies:
        c.wait()
    pltpu.sync_copy(rows_v, out_hbm.at[pl.ds(base, B_PER_W)])


def kernel(data, indices):
    idx = indices.astype(jnp.int32).reshape(NW, NCHUNK, CHUNK)
    return _sc_gather(data, idx)


# speedup vs baseline: 2.3349x; 2.3349x over previous
import functools

import jax
import jax.numpy as jnp
from jax import lax
from jax.experimental import pallas as pl
from jax.experimental.pallas import tpu as pltpu
from jax.experimental.pallas import tpu_sc as plsc

D = 64
B = 16384
NC, NS = 2, 16
NW = NC * NS
B_PER_W = B // NW
NBUF = 4
GROUP = 16
NGROUP = B_PER_W // GROUP

_LANES = None


def _extract_scalar(window, lane_iota, b):
    masked = jnp.where(lane_iota == b, window, jnp.int32(-2147483648))
    return lax.reduce_max(masked, axes=(0,))


@functools.partial(
    pl.kernel,
    mesh=plsc.VectorSubcoreMesh(core_axis_name="c", subcore_axis_name="s"),
    out_type=jax.ShapeDtypeStruct((B, D), jnp.float32),
    scratch_types=[
        pltpu.VMEM((B_PER_W,), jnp.int32),
        pltpu.VMEM((B_PER_W, D), jnp.float32),
        pltpu.VMEM((D, 128), jnp.float32),
        pltpu.VMEM((D, 128), jnp.float32),
        pltpu.VMEM((D, 128), jnp.float32),
        pltpu.VMEM((D, 128), jnp.float32),
        pltpu.SemaphoreType.DMA,
        pltpu.SemaphoreType.DMA,
        pltpu.SemaphoreType.DMA,
        pltpu.SemaphoreType.DMA,
    ],
    compiler_params=pltpu.CompilerParams(
        use_tc_tiling_on_sc=True, needs_layout_passes=False
    ),
)
def _sc_gather(tableT, idx_hbm, out_hbm,
               idx_v, rows_v, blk0, blk1, blk2, blk3, s0, s1, s2, s3):
    wid = lax.axis_index("s") * NC + lax.axis_index("c")
    base = wid * B_PER_W
    blks = [blk0, blk1, blk2, blk3]
    sems = [s0, s1, s2, s3]
    pltpu.sync_copy(idx_hbm.at[pl.ds(base, B_PER_W)], idx_v)

    lane_iota = lax.broadcasted_iota(jnp.int32, (16,), 0)

    def fire(window, b):
        idx = _extract_scalar(window, lane_iota, b)
        col = pl.multiple_of((idx >> 7) * 128, 128)
        pltpu.async_copy(
            tableT.at[:, pl.ds(col, 128)], blks[b % NBUF], sems[b % NBUF]
        )

    def drain_extract(window, b, g):
        idx = _extract_scalar(window, lane_iota, b)
        lane = idx & 127
        blk = blks[b % NBUF]
        pltpu.make_async_copy(
            tableT.at[:, pl.ds(0, 128)], blk, sems[b % NBUF]
        ).wait()
        j = g * GROUP + b
        for c4 in range(D // 16):
            row_idx = c4 * 16 + lane_iota
            vals = plsc.load_gather(blk, [row_idx, jnp.full((16,), 0, jnp.int32) + lane])
            plsc.store_scatter(
                rows_v, [jnp.full((16,), 0, jnp.int32) + j, row_idx], vals
            )

    def body(g, carry):
        window = idx_v[pl.ds(pl.multiple_of(g * GROUP, GROUP), GROUP)]
        for b in range(NBUF):
            fire(window, b)
        for b in range(GROUP - NBUF):
            drain_extract(window, b, g)
            fire(window, b + NBUF)
        for b in range(GROUP - NBUF, GROUP):
            drain_extract(window, b, g)
        return carry

    lax.fori_loop(0, NGROUP, body, 0)
    pltpu.sync_copy(rows_v, out_hbm.at[pl.ds(base, B_PER_W)])


def kernel(data, indices):
    idx = indices.astype(jnp.int32)
    return _sc_gather(data.T, idx)


# speedup vs baseline: 2.9542x; 1.2653x over previous
import functools

import jax
import jax.numpy as jnp
from jax import lax
from jax.experimental import pallas as pl
from jax.experimental.pallas import tpu as pltpu
from jax.experimental.pallas import tpu_sc as plsc

D = 64
B = 16384
NC, NS = 2, 16
NW = NC * NS
B_PER_W = B // NW
NBUF = 8
GROUP = 16
NGROUP = B_PER_W // GROUP
HALF = NGROUP // 2
CHUNK = HALF * GROUP


def _extract_scalar(window, lane_iota, b):
    masked = jnp.where(lane_iota == b, window, jnp.int32(-2147483648))
    return lax.reduce_max(masked, axes=(0,))


@functools.partial(
    pl.kernel,
    mesh=plsc.VectorSubcoreMesh(core_axis_name="c", subcore_axis_name="s"),
    out_type=jax.ShapeDtypeStruct((B, D), jnp.float32),
    scratch_types=[
        pltpu.VMEM((B_PER_W,), jnp.int32),
        pltpu.VMEM((CHUNK, D), jnp.float32),
        [pltpu.VMEM((D, 128), jnp.float32)] * NBUF,
        [pltpu.SemaphoreType.DMA] * NBUF,
    ],
    compiler_params=pltpu.CompilerParams(
        use_tc_tiling_on_sc=True, needs_layout_passes=False
    ),
)
def _sc_gather(tableT, idx_hbm, out_hbm, idx_v, rows_v, blks, sems):
    wid = lax.axis_index("s") * NC + lax.axis_index("c")
    base = wid * B_PER_W
    pltpu.sync_copy(idx_hbm.at[pl.ds(base, B_PER_W)], idx_v)

    lane_iota = lax.broadcasted_iota(jnp.int32, (16,), 0)

    def fire(window, b, slot):
        idx = _extract_scalar(window, lane_iota, b)
        col = pl.multiple_of((idx >> 7) * 128, 128)
        pltpu.async_copy(tableT.at[:, pl.ds(col, 128)], blks[slot], sems[slot])

    def drain_extract(window, b, slot, g):
        idx = _extract_scalar(window, lane_iota, b)
        lane = idx & 127
        blk = blks[slot]
        pltpu.make_async_copy(
            tableT.at[:, pl.ds(0, 128)], blk, sems[slot]
        ).wait()
        j = (g & (HALF - 1)) * GROUP + b
        jv = jnp.full((16,), 0, jnp.int32) + j
        lv = jnp.full((16,), 0, jnp.int32) + lane
        for c4 in range(D // 16):
            row_idx = c4 * 16 + lane_iota
            vals = plsc.load_gather(blk, [row_idx, lv])
            plsc.store_scatter(rows_v, [jv, row_idx], vals)

    def load_window(g):
        return idx_v[pl.ds(pl.multiple_of(g * GROUP, GROUP), GROUP)]

    def body(g, wcur):
        wnext = load_window(g + 1)
        for b in range(NBUF):
            drain_extract(wcur, b, b, g)
            fire(wcur, b + NBUF, b)
        for b in range(NBUF):
            drain_extract(wcur, b + NBUF, b, g)
            fire(wnext, b, b)
        return wnext

    w0 = load_window(0)
    for b in range(NBUF):
        fire(w0, b, b)

    w = lax.fori_loop(0, HALF, body, w0)
    pltpu.sync_copy(rows_v, out_hbm.at[pl.ds(base, CHUNK)])

    w = lax.fori_loop(HALF, NGROUP - 1, body, w)
    g = NGROUP - 1
    for b in range(NBUF):
        drain_extract(w, b, b, g)
        fire(w, b + NBUF, b)
    for b in range(NBUF):
        drain_extract(w, b + NBUF, b, g)
    pltpu.sync_copy(rows_v, out_hbm.at[pl.ds(base + CHUNK, CHUNK)])


def kernel(data, indices):
    idx = indices.astype(jnp.int32)
    return _sc_gather(data.T, idx)


# speedup vs baseline: 3.0193x; 1.0220x over previous
import functools

import jax
import jax.numpy as jnp
from jax import lax
from jax.experimental import pallas as pl
from jax.experimental.pallas import tpu as pltpu
from jax.experimental.pallas import tpu_sc as plsc

D = 64
B = 16384
NC, NS = 2, 16
NW = NC * NS
B_PER_W = B // NW
NBUF = 8
GROUP = 16
NGROUP = B_PER_W // GROUP
HALF = NGROUP // 2
CHUNK = HALF * GROUP


def _extract_scalar(window, lane_iota, b):
    masked = jnp.where(lane_iota == b, window, jnp.int32(-2147483648))
    return lax.reduce_max(masked, axes=(0,))


@functools.partial(
    pl.kernel,
    mesh=plsc.VectorSubcoreMesh(core_axis_name="c", subcore_axis_name="s"),
    out_type=jax.ShapeDtypeStruct((D, B), jnp.float32),
    scratch_types=[
        pltpu.VMEM((B_PER_W,), jnp.int32),
        pltpu.VMEM((D, CHUNK), jnp.float32),
        [pltpu.VMEM((D, 128), jnp.float32)] * NBUF,
        [pltpu.SemaphoreType.DMA] * NBUF,
    ],
    compiler_params=pltpu.CompilerParams(
        use_tc_tiling_on_sc=True, needs_layout_passes=False
    ),
)
def _sc_gather(tableT, idx_hbm, out_hbm, idx_v, rows_v, blks, sems):
    wid = lax.axis_index("s") * NC + lax.axis_index("c")
    base = wid * B_PER_W
    pltpu.sync_copy(idx_hbm.at[pl.ds(base, B_PER_W)], idx_v)

    lane_iota = lax.broadcasted_iota(jnp.int32, (16,), 0)

    def fire(window, b, slot):
        idx = _extract_scalar(window, lane_iota, b)
        col = pl.multiple_of((idx >> 7) * 128, 128)
        pltpu.async_copy(tableT.at[:, pl.ds(col, 128)], blks[slot], sems[slot])

    def drain_extract(window, b, slot, g):
        idx = _extract_scalar(window, lane_iota, b)
        lane = idx & 127
        blk = blks[slot]
        pltpu.make_async_copy(
            tableT.at[:, pl.ds(0, 128)], blk, sems[slot]
        ).wait()
        j = (g & (HALF - 1)) * GROUP + b
        jv = jnp.full((16,), 0, jnp.int32) + j
        lv = jnp.full((16,), 0, jnp.int32) + lane
        for c4 in range(D // 16):
            row_idx = c4 * 16 + lane_iota
            vals = plsc.load_gather(blk, [row_idx, lv])
            plsc.store_scatter(rows_v, [row_idx, jv], vals)

    def load_window(g):
        return idx_v[pl.ds(pl.multiple_of(g * GROUP, GROUP), GROUP)]

    def body(g, wcur):
        wnext = load_window(g + 1)
        for b in range(NBUF):
            drain_extract(wcur, b, b, g)
            fire(wcur, b + NBUF, b)
        for b in range(NBUF):
            drain_extract(wcur, b + NBUF, b, g)
            fire(wnext, b, b)
        return wnext

    w0 = load_window(0)
    for b in range(NBUF):
        fire(w0, b, b)

    w = lax.fori_loop(0, HALF, body, w0)
    pltpu.sync_copy(rows_v, out_hbm.at[:, pl.ds(base, CHUNK)])

    w = lax.fori_loop(HALF, NGROUP - 1, body, w)
    g = NGROUP - 1
    for b in range(NBUF):
        drain_extract(w, b, b, g)
        fire(w, b + NBUF, b)
    for b in range(NBUF):
        drain_extract(w, b + NBUF, b, g)
    pltpu.sync_copy(rows_v, out_hbm.at[:, pl.ds(base + CHUNK, CHUNK)])


def kernel(data, indices):
    idx = indices.astype(jnp.int32)
    return _sc_gather(data.T, idx).T
